# baseline (speedup 1.0000x reference)
import functools

import jax
import jax.numpy as jnp
from jax import lax
from jax.experimental import pallas as pl
from jax.experimental.pallas import tpu as pltpu
from jax.experimental.pallas import tpu_sc as plsc

_F = 26
_DP = 16
_CB = 2
_SB = 64


def _sc_gather(tables, idx3):
    B = idx3.shape[0] * idx3.shape[1] * idx3.shape[2] // 32
    D = tables.shape[2]
    mesh = plsc.VectorSubcoreMesh(core_axis_name="c", subcore_axis_name="s")
    NC = mesh.num_cores
    NW = NC * mesh.num_subcores
    NB = B // NW
    NR = _CB * _F
    NI = NB * 32 // 128

    @functools.partial(
        pl.kernel,
        out_type=jax.ShapeDtypeStruct((B, _F * _DP), jnp.float32),
        mesh=mesh,
        scratch_types=[
            pltpu.VMEM((NI, 128), jnp.int32),
            pltpu.VMEM((NR, 8, D), jnp.float32),
            pltpu.VMEM((_SB, _F * _DP), jnp.float32),
            pltpu.SemaphoreType.DMA,
        ],
        compiler_params=pltpu.CompilerParams(needs_layout_passes=False),
    )
    def gather_kernel(table_hbm, idx_hbm, out_hbm, idx_v, ring, stage, sem):
        wid = lax.axis_index("s") * NC + lax.axis_index("c")
        b0 = wid * NB
        pltpu.sync_copy(idx_hbm.at[wid], idx_v)
        lane16 = jax.lax.iota(jnp.int32, 16)
        claneD = jnp.minimum(lane16, D - 1)

        def chunk_body(c, carry):
            rows = []
            for i in range(_CB):
                bl = c * _CB + i
                for fw in (0, 16):
                    v16 = idx_v[bl // 4, pl.ds((bl % 4) * 32 + fw, 16)]
                    for l in range(16):
                        f = fw + l
                        if f >= _F:
                            break
                        row = v16[l]
                        rg = pl.multiple_of((row // 8) * 8, 8)
                        slot = i * _F + f
                        pltpu.async_copy(
                            table_hbm.at[f, pl.ds(rg, 8)],
                            ring.at[slot],
                            sem,
                        )
                        rows.append((slot, row - rg, i, f))
            for _ in range(NR):
                pltpu.make_async_copy(
                    table_hbm.at[0, pl.ds(0, 8)], ring.at[0], sem
                ).wait()
            si0 = (c % (_SB // _CB)) * _CB
            for slot, rm, i, f in rows:
                slot_v = jnp.full((16,), slot, dtype=jnp.int32)
                rm_v = jnp.broadcast_to(rm.astype(jnp.int32), (16,))
                v = plsc.load_gather(ring, [slot_v, rm_v, claneD])
                stage[si0 + i, pl.ds(f * _DP, _DP)] = v
            @pl.when(c % (_SB // _CB) == (_SB // _CB) - 1)
            def _():
                sb = (c // (_SB // _CB)) * _SB
                pltpu.sync_copy(stage, out_hbm.at[pl.ds(b0 + sb, _SB)])

            return carry

        lax.fori_loop(0, NB // _CB, chunk_body, 0)

    return gather_kernel(tables, idx3)


def _tc_matmul(emb16, W16, b2):
    B, K = emb16.shape
    H = W16.shape[1]
    BM = 1024

    def mm(emb_ref, w_ref, b_ref, out_ref):
        lane = lax.broadcasted_iota(jnp.int32, (BM, K), 1)
        e = jnp.where(lane % _DP < 10, emb_ref[...], 0.0)
        out_ref[...] = (
            jnp.dot(e, w_ref[...], preferred_element_type=jnp.float32) + b_ref[...]
        )

    return pl.pallas_call(
        mm,
        grid=(B // BM,),
        in_specs=[
            pl.BlockSpec((BM, K), lambda i: (i, 0)),
            pl.BlockSpec((K, H), lambda i: (0, 0)),
            pl.BlockSpec((1, H), lambda i: (0, 0)),
        ],
        out_specs=pl.BlockSpec((BM, H), lambda i: (i, 0)),
        out_shape=jax.ShapeDtypeStruct((B, H), jnp.float32),
    )(emb16, W16, b2)


def kernel(cov, tables, W, b):
    B, F = cov.shape
    _, V, D = tables.shape
    H = W.shape[1]
    idx2 = jnp.pad(cov.astype(jnp.int32), ((0, 0), (0, 32 - F)))
    idx3 = idx2.reshape(32, (B // 32) * 32 // 128, 128)

    emb16 = _sc_gather(tables, idx3)

    W16 = jnp.pad(W.reshape(F, D, H), ((0, 0), (0, _DP - D), (0, 0)))
    W16 = W16.reshape(F * _DP, H)
    return _tc_matmul(emb16, W16, b.reshape(1, H))


# speedup vs baseline: 1.0312x; 1.0312x over previous
import functools

import jax
import jax.numpy as jnp
from jax import lax
from jax.experimental import pallas as pl
from jax.experimental.pallas import tpu as pltpu
from jax.experimental.pallas import tpu_sc as plsc

_F = 26
_DP = 16
_SB = 64


def _sc_gather(tables, idx3, drain):
    B = idx3.shape[0] * idx3.shape[1] * idx3.shape[2] // 32
    D = tables.shape[2]
    mesh = plsc.VectorSubcoreMesh(core_axis_name="c", subcore_axis_name="s")
    NC = mesh.num_cores
    NW = NC * mesh.num_subcores
    NB = B // NW
    NI = NB * 32 // 128

    @functools.partial(
        pl.kernel,
        out_type=jax.ShapeDtypeStruct((B, _F * _DP), jnp.float32),
        mesh=mesh,
        scratch_types=[
            pltpu.VMEM((NI, 128), jnp.int32),
            pltpu.VMEM((2, _F, 8, D), jnp.float32),
            pltpu.VMEM((_SB, _F * _DP), jnp.float32),
            pltpu.SemaphoreType.DMA,
            pltpu.SemaphoreType.DMA,
        ],
        compiler_params=pltpu.CompilerParams(needs_layout_passes=False),
    )
    def gather_kernel(
        table_hbm, idx_hbm, drain_hbm, out_hbm, idx_v, ring, stage, sem0, sem1
    ):
        wid = lax.axis_index("s") * NC + lax.axis_index("c")
        b0 = wid * NB
        pltpu.sync_copy(idx_hbm.at[wid], idx_v)
        lane16 = jax.lax.iota(jnp.int32, 16)
        claneD = jnp.minimum(lane16, D - 1)
        sems = (sem0, sem1)

        def _windows(c):
            return (
                idx_v[c // 4, pl.ds((c % 4) * 32, 16)],
                idx_v[c // 4, pl.ds((c % 4) * 32 + 16, 16)],
            )

        def fire(c, rb, sem):
            ws = _windows(c)
            for f in range(_F):
                row = ws[f // 16][f % 16]
                rg = pl.multiple_of((row // 8) * 8, 8)
                pltpu.async_copy(
                    table_hbm.at[f, pl.ds(rg, 8)], ring.at[rb, f], sem
                )

        def extract(c, rb):
            si = c % _SB
            ws = _windows(c)
            for f in range(_F):
                row = ws[f // 16][f % 16]
                rm = row - (row // 8) * 8
                rb_v = jnp.full((16,), rb, dtype=jnp.int32)
                f_v = jnp.full((16,), f, dtype=jnp.int32)
                rm_v = jnp.broadcast_to(rm.astype(jnp.int32), (16,))
                v = plsc.load_gather(ring, [rb_v, f_v, rm_v, claneD])
                stage[si, pl.ds(f * _DP, _DP)] = v

        def drain_wait(rb, sem):
            pltpu.make_async_copy(drain_hbm, ring.at[rb], sem).wait()

        def flush(c):
            @pl.when(c % _SB == _SB - 1)
            def _():
                start = pl.multiple_of(b0 + (c - (_SB - 1)), _SB)
                pltpu.sync_copy(stage, out_hbm.at[pl.ds(start, _SB)])

        fire(0, 0, sem0)
        fire(1, 1, sem1)

        def body(cc, carry):
            for k in range(2):
                c = cc * 2 + k
                drain_wait(k, sems[k])
                extract(c, k)

                @pl.when(c + 2 < NB)
                def _():
                    fire(c + 2, k, sems[k])

                flush(c)
            return carry

        lax.fori_loop(0, NB // 2, body, 0)

    return gather_kernel(tables, idx3, drain)


def _tc_matmul(emb16, W16, b2):
    B, K = emb16.shape
    H = W16.shape[1]
    BM = 1024

    def mm(emb_ref, w_ref, b_ref, out_ref):
        lane = lax.broadcasted_iota(jnp.int32, (BM, K), 1)
        e = jnp.where(lane % _DP < 10, emb_ref[...], 0.0)
        out_ref[...] = (
            jnp.dot(e, w_ref[...], preferred_element_type=jnp.float32) + b_ref[...]
        )

    return pl.pallas_call(
        mm,
        grid=(B // BM,),
        in_specs=[
            pl.BlockSpec((BM, K), lambda i: (i, 0)),
            pl.BlockSpec((K, H), lambda i: (0, 0)),
            pl.BlockSpec((1, H), lambda i: (0, 0)),
        ],
        out_specs=pl.BlockSpec((BM, H), lambda i: (i, 0)),
        out_shape=jax.ShapeDtypeStruct((B, H), jnp.float32),
    )(emb16, W16, b2)


def kernel(cov, tables, W, b):
    B, F = cov.shape
    _, V, D = tables.shape
    H = W.shape[1]
    idx2 = jnp.pad(cov.astype(jnp.int32), ((0, 0), (0, 32 - F)))
    idx3 = idx2.reshape(32, (B // 32) * 32 // 128, 128)
    drain = jnp.zeros((F, 8, D), dtype=jnp.float32)

    emb16 = _sc_gather(tables, idx3, drain)

    W16 = jnp.pad(W.reshape(F, D, H), ((0, 0), (0, _DP - D), (0, 0)))
    W16 = W16.reshape(F * _DP, H)
    return _tc_matmul(emb16, W16, b.reshape(1, H))
